# baseline (speedup 1.0000x reference)
import functools

import jax
import jax.numpy as jnp
from jax import lax
from jax.experimental import pallas as pl
from jax.experimental.pallas import tpu as pltpu
from jax.experimental.pallas import tpu_sc as plsc

B = 16384
K = 20
D = 32
SLOTS = K + 1

NC = 2
NS = 16
NW = NC * NS
BPW = B // NW
CB = 32
NCHUNK = BPW // CB
ROWS_PER_CHUNK = CB * K
GATHER = 128
NG_NEG = ROWS_PER_CHUNK // GATHER
NG_POS = BPW // GATHER


def _sc_dots(v_i, v_j, ns_flat, w_node, w_sel):
    mesh = plsc.VectorSubcoreMesh(core_axis_name="c", subcore_axis_name="s")

    @functools.partial(
        pl.kernel,
        mesh=mesh,
        out_type=jax.ShapeDtypeStruct((B * SLOTS,), jnp.float32),
        scratch_types=[
            pltpu.VMEM((BPW,), jnp.int32),
            pltpu.VMEM((BPW,), jnp.int32),
            pltpu.VMEM((BPW * K,), jnp.int32),
            pltpu.VMEM((BPW, D), jnp.float32),
            pltpu.VMEM((BPW, D), jnp.float32),
            pltpu.VMEM((ROWS_PER_CHUNK, D), jnp.float32),
            pltpu.VMEM((BPW * SLOTS,), jnp.float32),
            pltpu.SemaphoreType.DMA,
        ],
        compiler_params=pltpu.CompilerParams(needs_layout_passes=False, use_tc_tiling_on_sc=False),
    )
    def k(vi_hbm, vj_hbm, ns_hbm, wn_hbm, ws_hbm, out_hbm,
          vi_idx, vj_idx, ns_idx, vi_rows, vj_rows, negbuf, dots, sem):
        wid = lax.axis_index("s") * NC + lax.axis_index("c")
        base = wid * BPW

        zeros = jnp.zeros((16,), jnp.float32)

        def z_body(i, carry):
            dots[pl.ds(i * 16, 16)] = zeros
            return carry

        lax.fori_loop(0, BPW * SLOTS // 16, z_body, 0)

        pltpu.sync_copy(vi_hbm.at[pl.ds(base, BPW)], vi_idx)
        pltpu.sync_copy(vj_hbm.at[pl.ds(base, BPW)], vj_idx)
        pltpu.sync_copy(ns_hbm.at[pl.ds(base * K, BPW * K)], ns_idx)

        cps = []
        for g in range(NG_POS):
            sl = pl.ds(g * GATHER, GATHER)
            cps.append(pltpu.async_copy(wn_hbm.at[vi_idx.at[sl]],
                                        vi_rows.at[sl], sem))
            cps.append(pltpu.async_copy(ws_hbm.at[vj_idx.at[sl]],
                                        vj_rows.at[sl], sem))
        for cp in cps:
            cp.wait()

        def chunk_body(c, carry):
            copies = []
            for g in range(NG_NEG):
                copies.append(pltpu.async_copy(
                    ws_hbm.at[ns_idx.at[pl.ds(c * ROWS_PER_CHUNK + g * GATHER,
                                              GATHER)]],
                    negbuf.at[pl.ds(g * GATHER, GATHER)], sem))
            for cp in copies:
                cp.wait()

            def b_body(bl, carry2):
                gb = c * CB + bl
                obase = gb * SLOTS
                a0 = vi_rows[gb, pl.ds(0, 16)]
                a1 = vi_rows[gb, pl.ds(16, 16)]
                h0 = vj_rows[gb, pl.ds(0, 16)]
                h1 = vj_rows[gb, pl.ds(16, 16)]
                p = a0 * h0 + a1 * h1
                plsc.addupdate_scatter(
                    dots, [jnp.full((16,), obase, jnp.int32)], p)
                for kk in range(K):
                    j = bl * K + kk
                    n0 = negbuf[j, pl.ds(0, 16)]
                    n1 = negbuf[j, pl.ds(16, 16)]
                    p = a0 * n0 + a1 * n1
                    plsc.addupdate_scatter(
                        dots, [jnp.full((16,), obase + 1 + kk, jnp.int32)], -p)
                return carry2

            return lax.fori_loop(0, CB, b_body, carry)

        lax.fori_loop(0, NCHUNK, chunk_body, 0)

        pltpu.sync_copy(dots, out_hbm.at[pl.ds(base * SLOTS, BPW * SLOTS)])

    return k(v_i, v_j, ns_flat, w_node, w_sel)


def _tc_loss(dots):
    x = dots.reshape(B * SLOTS // 128, 128)

    def body(x_ref, o_ref):
        v = x_ref[...]
        o_ref[0, 0] = -jnp.sum(jax.nn.log_sigmoid(v)) / B

    out = pl.pallas_call(
        body,
        out_shape=jax.ShapeDtypeStruct((1, 1), jnp.float32),
        out_specs=pl.BlockSpec(memory_space=pltpu.SMEM),
    )(x)
    return out[0, 0]


def kernel(v_i, v_j, negsamples, order, W_node, W_ctx):
    v_i = v_i.astype(jnp.int32)
    v_j = v_j.astype(jnp.int32)
    ns_flat = negsamples.reshape(-1).astype(jnp.int32)
    dots = lax.cond(
        order == 1,
        lambda wn, wc: _sc_dots(v_i, v_j, ns_flat, wn, wn),
        lambda wn, wc: _sc_dots(v_i, v_j, ns_flat, wn, wc),
        W_node, W_ctx,
    )
    return _tc_loss(dots)


# speedup vs baseline: 1.0806x; 1.0806x over previous
import functools

import jax
import jax.numpy as jnp
from jax import lax
from jax.experimental import pallas as pl
from jax.experimental.pallas import tpu as pltpu
from jax.experimental.pallas import tpu_sc as plsc

N_NODES_ = 1000000
B = 16384
K = 20
D = 32
W = 128
SLOTS = K + 1

NC = 2
NS = 16
NW = NC * NS
BPW = B // NW
CB = 8
NCHUNK = BPW // CB
ROWS_PER_CHUNK = CB * K
GATHER = 128
NEG_GATHERS = ((0, 128), (128, 32))
NG_POS = BPW // GATHER


def _sc_dots(v_i, v_j, ns_flat, w_cat):
    mesh = plsc.VectorSubcoreMesh(core_axis_name="c", subcore_axis_name="s")

    @functools.partial(
        pl.kernel,
        mesh=mesh,
        out_type=jax.ShapeDtypeStruct((B * SLOTS,), jnp.float32),
        scratch_types=[
            pltpu.VMEM((BPW,), jnp.int32),
            pltpu.VMEM((BPW,), jnp.int32),
            pltpu.VMEM((BPW * K,), jnp.int32),
            pltpu.VMEM((BPW, 2 * D), jnp.float32),
            pltpu.VMEM((ROWS_PER_CHUNK, W), jnp.float32),
            pltpu.VMEM((BPW * SLOTS,), jnp.float32),
            pltpu.SemaphoreType.DMA,
        ],
        compiler_params=pltpu.CompilerParams(needs_layout_passes=False),
    )
    def k(vi_hbm, vj_hbm, ns_hbm, wc_hbm, out_hbm,
          vi_idx, vj_idx, ns_idx, vivj, negbuf, dots, sem):
        wid = lax.axis_index("s") * NC + lax.axis_index("c")
        base = wid * BPW

        zeros = jnp.zeros((16,), jnp.float32)

        def z_body(i, carry):
            dots[pl.ds(i * 16, 16)] = zeros
            return carry

        lax.fori_loop(0, BPW * SLOTS // 16, z_body, 0)

        pltpu.sync_copy(vi_hbm.at[pl.ds(base, BPW)], vi_idx)
        pltpu.sync_copy(vj_hbm.at[pl.ds(base, BPW)], vj_idx)
        pltpu.sync_copy(ns_hbm.at[pl.ds(base * K, BPW * K)], ns_idx)

        stage = negbuf.at[pl.ds(0, GATHER)]

        def pos_body(g, carry):
            sl = pl.ds(g * GATHER, GATHER)
            pltpu.async_copy(wc_hbm.at[vi_idx.at[sl]], stage, sem).wait()

            def ci_body(r, carry2):
                gb = g * GATHER + r
                vivj[gb, pl.ds(0, 16)] = negbuf[r, pl.ds(0, 16)]
                vivj[gb, pl.ds(16, 16)] = negbuf[r, pl.ds(16, 16)]
                return carry2

            lax.fori_loop(0, GATHER, ci_body, carry)
            pltpu.async_copy(wc_hbm.at[vj_idx.at[sl]], stage, sem).wait()

            def cj_body(r, carry2):
                gb = g * GATHER + r
                vivj[gb, pl.ds(32, 16)] = negbuf[r, pl.ds(32, 16)]
                vivj[gb, pl.ds(48, 16)] = negbuf[r, pl.ds(48, 16)]
                return carry2

            return lax.fori_loop(0, GATHER, cj_body, carry)

        lax.fori_loop(0, NG_POS, pos_body, 0)

        def chunk_body(c, carry):
            copies = []
            for off, sz in NEG_GATHERS:
                copies.append(pltpu.async_copy(
                    wc_hbm.at[ns_idx.at[pl.ds(c * ROWS_PER_CHUNK + off, sz)]],
                    negbuf.at[pl.ds(off, sz)], sem))
            for cp in copies:
                cp.wait()

            def b_body(bl, carry2):
                gb = c * CB + bl
                obase = gb * SLOTS
                a0 = vivj[gb, pl.ds(0, 16)]
                a1 = vivj[gb, pl.ds(16, 16)]
                h0 = vivj[gb, pl.ds(32, 16)]
                h1 = vivj[gb, pl.ds(48, 16)]
                p = a0 * h0 + a1 * h1
                plsc.addupdate_scatter(
                    dots, [jnp.full((16,), obase, jnp.int32)], p)
                for kk in range(K):
                    j = bl * K + kk
                    n0 = negbuf[j, pl.ds(32, 16)]
                    n1 = negbuf[j, pl.ds(48, 16)]
                    p = a0 * n0 + a1 * n1
                    plsc.addupdate_scatter(
                        dots, [jnp.full((16,), obase + 1 + kk, jnp.int32)], -p)
                return carry2

            return lax.fori_loop(0, CB, b_body, carry)

        lax.fori_loop(0, NCHUNK, chunk_body, 0)

        pltpu.sync_copy(dots, out_hbm.at[pl.ds(base * SLOTS, BPW * SLOTS)])

    return k(v_i, v_j, ns_flat, w_cat)


def _tc_loss(dots):
    x = dots.reshape(B * SLOTS // 128, 128)

    def body(x_ref, o_ref):
        v = x_ref[...]
        o_ref[0, 0] = -jnp.sum(jax.nn.log_sigmoid(v)) / B

    out = pl.pallas_call(
        body,
        out_shape=jax.ShapeDtypeStruct((1, 1), jnp.float32),
        out_specs=pl.BlockSpec(memory_space=pltpu.SMEM),
    )(x)
    return out[0, 0]


def kernel(v_i, v_j, negsamples, order, W_node, W_ctx):
    v_i = v_i.astype(jnp.int32)
    v_j = v_j.astype(jnp.int32)
    ns_flat = negsamples.reshape(-1).astype(jnp.int32)
    zpad = jnp.zeros((N_NODES_, W - 2 * D), jnp.float32)

    def br1(wn, wc):
        return _sc_dots(v_i, v_j, ns_flat,
                        jnp.concatenate([wn, wn, zpad], axis=1))

    def br2(wn, wc):
        return _sc_dots(v_i, v_j, ns_flat,
                        jnp.concatenate([wn, wc, zpad], axis=1))

    dots = lax.cond(order == 1, br1, br2, W_node, W_ctx)
    return _tc_loss(dots)


# speedup vs baseline: 1.1561x; 1.0699x over previous
import functools

import jax
import jax.numpy as jnp
from jax import lax
from jax.experimental import pallas as pl
from jax.experimental.pallas import tpu as pltpu
from jax.experimental.pallas import tpu_sc as plsc

N_NODES_ = 1000000
B = 16384
K = 20
D = 32
W = 128
SLOTS = K + 1

NC = 2
NS = 16
NW = NC * NS
BPW = B // NW
CB = 8
NCHUNK = BPW // CB
ROWS_PER_CHUNK = CB * K
GATHER = 128
NEG_GATHERS = ((0, 128), (128, 32))
NG_POS = BPW // GATHER


def _sc_dots(v_i, v_j, ns_flat, w_cat):
    mesh = plsc.VectorSubcoreMesh(core_axis_name="c", subcore_axis_name="s")

    @functools.partial(
        pl.kernel,
        mesh=mesh,
        out_type=jax.ShapeDtypeStruct((B * SLOTS,), jnp.float32),
        scratch_types=[
            pltpu.VMEM((BPW,), jnp.int32),
            pltpu.VMEM((BPW,), jnp.int32),
            pltpu.VMEM((BPW * K,), jnp.int32),
            pltpu.VMEM((BPW, 2 * D), jnp.float32),
            pltpu.VMEM((ROWS_PER_CHUNK, W), jnp.float32),
            pltpu.VMEM((ROWS_PER_CHUNK, W), jnp.float32),
            pltpu.VMEM((BPW * SLOTS,), jnp.float32),
            pltpu.SemaphoreType.DMA,
            pltpu.SemaphoreType.DMA,
        ],
        compiler_params=pltpu.CompilerParams(needs_layout_passes=False),
    )
    def k(vi_hbm, vj_hbm, ns_hbm, wc_hbm, out_hbm,
          vi_idx, vj_idx, ns_idx, vivj, negbuf0, negbuf1, dots, sem0, sem1):
        wid = lax.axis_index("s") * NC + lax.axis_index("c")
        base = wid * BPW
        bufs = (negbuf0, negbuf1)
        sems = (sem0, sem1)

        zeros = jnp.zeros((16,), jnp.float32)

        def z_body(i, carry):
            dots[pl.ds(i * 16, 16)] = zeros
            return carry

        lax.fori_loop(0, BPW * SLOTS // 16, z_body, 0)

        pltpu.sync_copy(vi_hbm.at[pl.ds(base, BPW)], vi_idx)
        pltpu.sync_copy(vj_hbm.at[pl.ds(base, BPW)], vj_idx)
        pltpu.sync_copy(ns_hbm.at[pl.ds(base * K, BPW * K)], ns_idx)

        def neg_copies(c, buf, sem):
            return [
                pltpu.make_async_copy(
                    wc_hbm.at[ns_idx.at[pl.ds(c * ROWS_PER_CHUNK + off, sz)]],
                    buf.at[pl.ds(off, sz)], sem)
                for off, sz in NEG_GATHERS
            ]

        stage = negbuf0.at[pl.ds(0, GATHER)]

        def pos_body(g, carry):
            sl = pl.ds(g * GATHER, GATHER)
            pltpu.async_copy(wc_hbm.at[vi_idx.at[sl]], stage, sem0).wait()

            def ci_body(r, carry2):
                gb = g * GATHER + r
                vivj[gb, pl.ds(0, 16)] = negbuf0[r, pl.ds(0, 16)]
                vivj[gb, pl.ds(16, 16)] = negbuf0[r, pl.ds(16, 16)]
                return carry2

            lax.fori_loop(0, GATHER, ci_body, carry)
            pltpu.async_copy(wc_hbm.at[vj_idx.at[sl]], stage, sem0).wait()

            def cj_body(r, carry2):
                gb = g * GATHER + r
                vivj[gb, pl.ds(32, 16)] = negbuf0[r, pl.ds(32, 16)]
                vivj[gb, pl.ds(48, 16)] = negbuf0[r, pl.ds(48, 16)]
                return carry2

            return lax.fori_loop(0, GATHER, cj_body, carry)

        lax.fori_loop(0, NG_POS, pos_body, 0)

        for cp in neg_copies(0, negbuf0, sem0):
            cp.start()

        def compute_chunk(c, buf):
            def b_body(bl, carry2):
                gb = c * CB + bl
                obase = gb * SLOTS
                a0 = vivj[gb, pl.ds(0, 16)]
                a1 = vivj[gb, pl.ds(16, 16)]
                h0 = vivj[gb, pl.ds(32, 16)]
                h1 = vivj[gb, pl.ds(48, 16)]
                p = a0 * h0 + a1 * h1
                plsc.addupdate_scatter(
                    dots, [jnp.full((16,), obase, jnp.int32)], p)
                for kk in range(K):
                    j = bl * K + kk
                    n0 = buf[j, pl.ds(32, 16)]
                    n1 = buf[j, pl.ds(48, 16)]
                    p = a0 * n0 + a1 * n1
                    plsc.addupdate_scatter(
                        dots, [jnp.full((16,), obase + 1 + kk, jnp.int32)], -p)
                return carry2

            lax.fori_loop(0, CB, b_body, 0)

        def pair_body(i, carry):
            for par in (0, 1):
                c = i * 2 + par
                nxt = (par + 1) % 2

                @pl.when(c + 1 < NCHUNK)
                def _():
                    for cp in neg_copies(c + 1, bufs[nxt], sems[nxt]):
                        cp.start()

                for cp in neg_copies(c, bufs[par], sems[par]):
                    cp.wait()
                compute_chunk(c, bufs[par])
            return carry

        lax.fori_loop(0, NCHUNK // 2, pair_body, 0)

        pltpu.sync_copy(dots, out_hbm.at[pl.ds(base * SLOTS, BPW * SLOTS)])

    return k(v_i, v_j, ns_flat, w_cat)


def _tc_loss(dots):
    x = dots.reshape(B * SLOTS // 128, 128)

    def body(x_ref, o_ref):
        v = x_ref[...]
        o_ref[0, 0] = -jnp.sum(jax.nn.log_sigmoid(v)) / B

    out = pl.pallas_call(
        body,
        out_shape=jax.ShapeDtypeStruct((1, 1), jnp.float32),
        out_specs=pl.BlockSpec(memory_space=pltpu.SMEM),
    )(x)
    return out[0, 0]


def kernel(v_i, v_j, negsamples, order, W_node, W_ctx):
    v_i = v_i.astype(jnp.int32)
    v_j = v_j.astype(jnp.int32)
    ns_flat = negsamples.reshape(-1).astype(jnp.int32)
    zpad = jnp.zeros((N_NODES_, W - 2 * D), jnp.float32)

    def br1(wn, wc):
        return _sc_dots(v_i, v_j, ns_flat,
                        jnp.concatenate([wn, wn, zpad], axis=1))

    def br2(wn, wc):
        return _sc_dots(v_i, v_j, ns_flat,
                        jnp.concatenate([wn, wc, zpad], axis=1))

    dots = lax.cond(order == 1, br1, br2, W_node, W_ctx)
    return _tc_loss(dots)
